# baseline (device time: 11397 ns/iter reference)
import jax
import jax.numpy as jnp
from jax import lax
from jax.experimental import pallas as pl
from jax.experimental.pallas import tpu as pltpu

N_DEV = 4
B = 2
SQ = 256
SKV = 256
HQ = 4
DH = 64
D = 512
HD = HQ * DH
HALO = 128
WIN = 128


def _body(x_hbm, wq_hbm, kt_hbm, vt_hbm, wo_hbm, out_hbm,
          x_vmem, wq_vmem, kt_f32, vt_f32, wo_vmem, out_vmem,
          kt_bf, vt_bf, kleft, kright, vleft, vright,
          send_sems, recv_sems, in_sems, out_sems):
    bf16 = jnp.bfloat16
    pos = lax.axis_index("i")
    left = (pos - 1) % N_DEV
    right = (pos + 1) % N_DEV

    fetches = [
        pltpu.make_async_copy(src, dst, in_sems.at[i])
        for i, (src, dst) in enumerate((
            (kt_hbm, kt_f32), (vt_hbm, vt_f32),
            (x_hbm, x_vmem), (wq_hbm, wq_vmem), (wo_hbm, wo_vmem),
        ))
    ]
    for f in fetches:
        f.start()

    fetches[0].wait()
    fetches[1].wait()
    for b in range(B):
        for half in range(2):
            cols = slice(half * HALO, (half + 1) * HALO)
            kt_bf[2 * b + half] = (
                kt_f32[b, :, :, cols].reshape(HD, HALO).astype(bf16))
            vt_bf[2 * b + half] = (
                vt_f32[b, :, :, cols].reshape(HD, HALO).astype(bf16))

    barrier_sem = pltpu.get_barrier_semaphore()
    for nbr in (left, right):
        pl.semaphore_signal(
            barrier_sem, inc=1,
            device_id=(nbr,), device_id_type=pl.DeviceIdType.MESH,
        )
    pl.semaphore_wait(barrier_sem, 2)

    k_rdmas, v_rdmas = [], []
    sem_idx = 0
    for rdma_list, buf, dst_l, dst_r in (
        (k_rdmas, kt_bf, kleft, kright),
        (v_rdmas, vt_bf, vleft, vright),
    ):
        for b in range(B):
            for src, dst, tgt in (
                (buf.at[2 * b + 1], dst_l.at[b], right),
                (buf.at[2 * b], dst_r.at[b], left),
            ):
                rdma = pltpu.make_async_remote_copy(
                    src_ref=src, dst_ref=dst,
                    send_sem=send_sems.at[sem_idx],
                    recv_sem=recv_sems.at[sem_idx],
                    device_id=(tgt,), device_id_type=pl.DeviceIdType.MESH,
                )
                rdma.start()
                rdma_list.append(rdma)
                sem_idx += 1


    fetches[2].wait()
    fetches[3].wait()
    wq = wq_vmem[...].astype(bf16)
    q = []
    for b in range(B):
        qb = lax.dot_general(
            x_vmem[b].astype(bf16), wq,
            (((1,), (0,)), ((), ())), preferred_element_type=jnp.float32,
        )
        q.append((qb * 0.125).astype(bf16))

    KW = SKV + 2 * HALO
    i2 = lax.broadcasted_iota(jnp.int32, (SQ, KW), 0)
    j2 = lax.broadcasted_iota(jnp.int32, (SQ, KW), 1)
    ki = pos * SKV - HALO + j2
    mask = (jnp.abs(i2 + HALO - j2) <= WIN) & (ki >= 0) & (ki < N_DEV * SKV)
    bias = jnp.where(mask, 0.0, -1e9).astype(jnp.float32)

    fetches[4].wait()
    wo = wo_vmem[...].astype(bf16)

    out_copies = []
    for b in range(B):
        for rdma in k_rdmas[2 * b:2 * b + 2]:
            rdma.wait_recv()
        ws = []
        for h in range(HQ):
            rows = slice(h * DH, (h + 1) * DH)
            khT = jnp.concatenate(
                [kleft[b][rows], kt_bf[2 * b][rows],
                 kt_bf[2 * b + 1][rows], kright[b][rows]], axis=1)
            s = lax.dot_general(
                q[b][:, rows], khT, (((1,), (0,)), ((), ())),
                preferred_element_type=jnp.float32,
            ) + bias
            e = jnp.exp(s)
            ws.append(
                (e * (1.0 / jnp.sum(e, axis=1, keepdims=True))).astype(bf16))
        for rdma in v_rdmas[2 * b:2 * b + 2]:
            rdma.wait_recv()
        ctx_heads = []
        for h in range(HQ):
            rows = slice(h * DH, (h + 1) * DH)
            vhT = jnp.concatenate(
                [vleft[b][rows], vt_bf[2 * b][rows],
                 vt_bf[2 * b + 1][rows], vright[b][rows]], axis=1)
            ctx_heads.append(lax.dot_general(
                ws[h], vhT, (((1,), (1,)), ((), ())),
                preferred_element_type=jnp.float32,
            ))
        ctx_b = jnp.concatenate(ctx_heads, axis=1).astype(bf16)
        out_vmem[b] = lax.dot_general(
            ctx_b, wo,
            (((1,), (0,)), ((), ())), preferred_element_type=jnp.float32,
        ).astype(bf16)
        oc = pltpu.make_async_copy(
            out_vmem.at[b], out_hbm.at[b], out_sems.at[b])
        oc.start()
        out_copies.append(oc)

    for rdma in k_rdmas + v_rdmas:
        rdma.wait_send()
    for oc in out_copies:
        oc.wait()


def kernel(x, Wq, K_ext, V_ext, Wo):
    kt = jnp.transpose(K_ext, (0, 2, 3, 1))
    vt = jnp.transpose(V_ext, (0, 2, 3, 1))
    hbm = pltpu.MemorySpace.HBM
    args = [pltpu.with_memory_space_constraint(a, hbm)
            for a in (x, Wq, kt, vt, Wo)]
    halo = pltpu.VMEM((B, HD, HALO), jnp.bfloat16)
    return pl.pallas_call(
        _body,
        out_shape=jax.ShapeDtypeStruct((B, SQ, D), jnp.bfloat16),
        in_specs=[pl.BlockSpec(memory_space=hbm)] * 5,
        out_specs=pl.BlockSpec(memory_space=hbm),
        scratch_shapes=[
            pltpu.VMEM((B, SQ, D), jnp.float32),
            pltpu.VMEM((D, HD), jnp.float32),
            pltpu.VMEM((B, HQ, DH, SKV), jnp.float32),
            pltpu.VMEM((B, HQ, DH, SKV), jnp.float32),
            pltpu.VMEM((HD, D), jnp.float32),
            pltpu.VMEM((B, SQ, D), jnp.bfloat16),
            pltpu.VMEM((B * 2, HD, HALO), jnp.bfloat16),
            pltpu.VMEM((B * 2, HD, HALO), jnp.bfloat16),
            halo, halo, halo, halo,
            pltpu.SemaphoreType.DMA((8,)),
            pltpu.SemaphoreType.DMA((8,)),
            pltpu.SemaphoreType.DMA((5,)),
            pltpu.SemaphoreType.DMA((B,)),
        ],
        compiler_params=pltpu.CompilerParams(collective_id=0),
    )(*args)


# device time: 9545 ns/iter; 1.1940x vs baseline; 1.1940x over previous
import jax
import jax.numpy as jnp
from jax import lax
from jax.experimental import pallas as pl
from jax.experimental.pallas import tpu as pltpu

N_DEV = 4
B = 2
SQ = 256
SKV = 256
HQ = 4
DH = 64
D = 512
HD = HQ * DH
HALO = 128
WIN = 128


def _body(x_hbm, wq_hbm, kt_hbm, vt_hbm, wo_hbm, out_hbm,
          x_vmem, wq_vmem, kt_f32, vt_f32, wo_vmem, out_vmem,
          kt_bf, vt_bf, kleft, kright, vleft, vright,
          send_sems, recv_sems, in_sems, out_sems):
    bf16 = jnp.bfloat16
    pos = lax.axis_index("i")
    left = (pos - 1) % N_DEV
    right = (pos + 1) % N_DEV

    fetches = [
        pltpu.make_async_copy(src, dst, in_sems.at[i])
        for i, (src, dst) in enumerate((
            (kt_hbm, kt_f32), (vt_hbm, vt_f32),
            (x_hbm, x_vmem), (wq_hbm, wq_vmem), (wo_hbm, wo_vmem),
        ))
    ]
    for f in fetches:
        f.start()

    barrier_sem = pltpu.get_barrier_semaphore()
    for nbr in (left, right):
        pl.semaphore_signal(
            barrier_sem, inc=1,
            device_id=(nbr,), device_id_type=pl.DeviceIdType.MESH,
        )
    pl.semaphore_wait(barrier_sem, 2)

    fetches[0].wait()
    fetches[1].wait()
    k_rdmas, v_rdmas = [], []
    sem_idx = 0
    for b in range(B):
        for rdma_list, f32buf, bfbuf, dst_l, dst_r in (
            (k_rdmas, kt_f32, kt_bf, kleft, kright),
            (v_rdmas, vt_f32, vt_bf, vleft, vright),
        ):
            for half in range(2):
                cols = slice(half * HALO, (half + 1) * HALO)
                bfbuf[2 * b + half] = (
                    f32buf[b, :, :, cols].reshape(HD, HALO).astype(bf16))
            for src, dst, tgt in (
                (bfbuf.at[2 * b + 1], dst_l.at[b], right),
                (bfbuf.at[2 * b], dst_r.at[b], left),
            ):
                rdma = pltpu.make_async_remote_copy(
                    src_ref=src, dst_ref=dst,
                    send_sem=send_sems.at[sem_idx],
                    recv_sem=recv_sems.at[sem_idx],
                    device_id=(tgt,), device_id_type=pl.DeviceIdType.MESH,
                )
                rdma.start()
                rdma_list.append(rdma)
                sem_idx += 1


    fetches[2].wait()
    fetches[3].wait()
    wq = wq_vmem[...].astype(bf16)
    q = []
    for b in range(B):
        qb = lax.dot_general(
            x_vmem[b].astype(bf16), wq,
            (((1,), (0,)), ((), ())), preferred_element_type=jnp.float32,
        )
        q.append((qb * 0.125).astype(bf16))

    KW = SKV + 2 * HALO
    i2 = lax.broadcasted_iota(jnp.int32, (SQ, KW), 0)
    j2 = lax.broadcasted_iota(jnp.int32, (SQ, KW), 1)
    ki = pos * SKV - HALO + j2
    mask = (jnp.abs(i2 + HALO - j2) <= WIN) & (ki >= 0) & (ki < N_DEV * SKV)
    bias = jnp.where(mask, 0.0, -1e4).astype(bf16)

    fetches[4].wait()
    wo = wo_vmem[...].astype(bf16)

    out_copies = []
    for b in range(B):
        for rdma in k_rdmas[2 * b:2 * b + 2]:
            rdma.wait_recv()
        ws = []
        for h in range(HQ):
            rows = slice(h * DH, (h + 1) * DH)
            khT = jnp.concatenate(
                [kleft[b][rows], kt_bf[2 * b][rows],
                 kt_bf[2 * b + 1][rows], kright[b][rows]], axis=1)
            s = lax.dot_general(
                q[b][:, rows], khT, (((1,), (0,)), ((), ())),
                preferred_element_type=jnp.float32,
            ).astype(bf16) + bias
            e = jnp.exp(s)
            ws.append((
                e,
                1.0 / jnp.sum(e, axis=1, keepdims=True,
                              dtype=jnp.float32)))
        for rdma in v_rdmas[2 * b:2 * b + 2]:
            rdma.wait_recv()
        ctx_heads = []
        for h in range(HQ):
            rows = slice(h * DH, (h + 1) * DH)
            vhT = jnp.concatenate(
                [vleft[b][rows], vt_bf[2 * b][rows],
                 vt_bf[2 * b + 1][rows], vright[b][rows]], axis=1)
            e, recip = ws[h]
            ctx_heads.append(lax.dot_general(
                e, vhT, (((1,), (1,)), ((), ())),
                preferred_element_type=jnp.float32,
            ) * recip)
        ctx_b = jnp.concatenate(ctx_heads, axis=1).astype(bf16)
        out_vmem[b] = lax.dot_general(
            ctx_b, wo,
            (((1,), (0,)), ((), ())), preferred_element_type=jnp.float32,
        ).astype(bf16)
        oc = pltpu.make_async_copy(
            out_vmem.at[b], out_hbm.at[b], out_sems.at[b])
        oc.start()
        out_copies.append(oc)

    for rdma in k_rdmas + v_rdmas:
        rdma.wait_send()
    for oc in out_copies:
        oc.wait()


def kernel(x, Wq, K_ext, V_ext, Wo):
    kt = jnp.transpose(K_ext, (0, 2, 3, 1))
    vt = jnp.transpose(V_ext, (0, 2, 3, 1))
    hbm = pltpu.MemorySpace.HBM
    args = [pltpu.with_memory_space_constraint(a, hbm)
            for a in (x, Wq, kt, vt, Wo)]
    halo = pltpu.VMEM((B, HD, HALO), jnp.bfloat16)
    return pl.pallas_call(
        _body,
        out_shape=jax.ShapeDtypeStruct((B, SQ, D), jnp.bfloat16),
        in_specs=[pl.BlockSpec(memory_space=hbm)] * 5,
        out_specs=pl.BlockSpec(memory_space=hbm),
        scratch_shapes=[
            pltpu.VMEM((B, SQ, D), jnp.float32),
            pltpu.VMEM((D, HD), jnp.float32),
            pltpu.VMEM((B, HQ, DH, SKV), jnp.float32),
            pltpu.VMEM((B, HQ, DH, SKV), jnp.float32),
            pltpu.VMEM((HD, D), jnp.float32),
            pltpu.VMEM((B, SQ, D), jnp.bfloat16),
            pltpu.VMEM((B * 2, HD, HALO), jnp.bfloat16),
            pltpu.VMEM((B * 2, HD, HALO), jnp.bfloat16),
            halo, halo, halo, halo,
            pltpu.SemaphoreType.DMA((8,)),
            pltpu.SemaphoreType.DMA((8,)),
            pltpu.SemaphoreType.DMA((5,)),
            pltpu.SemaphoreType.DMA((B,)),
        ],
        compiler_params=pltpu.CompilerParams(collective_id=0),
    )(*args)


# device time: 9526 ns/iter; 1.1964x vs baseline; 1.0020x over previous
import jax
import jax.numpy as jnp
from jax import lax
from jax.experimental import pallas as pl
from jax.experimental.pallas import tpu as pltpu

N_DEV = 4
B = 2
SQ = 256
SKV = 256
HQ = 4
DH = 64
D = 512
HD = HQ * DH
HALO = 128
WIN = 128


def _body(x_hbm, wq_hbm, kt_hbm, vt_hbm, wo_hbm, out_hbm,
          x_vmem, wq_vmem, kt_f32, vt_f32, wo_vmem, out_vmem,
          kt_bf, vt_bf, kleft, kright, vleft, vright,
          send_sems, recv_sems, in_sems, out_sems):
    bf16 = jnp.bfloat16
    pos = lax.axis_index("i")
    left = (pos - 1) % N_DEV
    right = (pos + 1) % N_DEV

    fetches = [
        pltpu.make_async_copy(src, dst, in_sems.at[i])
        for i, (src, dst) in enumerate((
            (kt_hbm, kt_f32), (vt_hbm, vt_f32),
            (x_hbm, x_vmem), (wq_hbm, wq_vmem), (wo_hbm, wo_vmem),
        ))
    ]
    for f in fetches:
        f.start()

    barrier_sem = pltpu.get_barrier_semaphore()
    for nbr in (left, right):
        pl.semaphore_signal(
            barrier_sem, inc=1,
            device_id=(nbr,), device_id_type=pl.DeviceIdType.MESH,
        )
    pl.semaphore_wait(barrier_sem, 2)

    fetches[0].wait()
    fetches[1].wait()
    k_rdmas, v_rdmas = [], []
    sem_idx = 0
    for b in range(B):
        for rdma_list, f32buf, bfbuf, dst_l, dst_r in (
            (k_rdmas, kt_f32, kt_bf, kleft, kright),
            (v_rdmas, vt_f32, vt_bf, vleft, vright),
        ):
            for half in range(2):
                cols = slice(half * HALO, (half + 1) * HALO)
                bfbuf[2 * b + half] = (
                    f32buf[b, :, :, cols].reshape(HD, HALO).astype(bf16))
            for src, dst, tgt in (
                (bfbuf.at[2 * b + 1], dst_l.at[b], right),
                (bfbuf.at[2 * b], dst_r.at[b], left),
            ):
                rdma = pltpu.make_async_remote_copy(
                    src_ref=src, dst_ref=dst,
                    send_sem=send_sems.at[sem_idx],
                    recv_sem=recv_sems.at[sem_idx],
                    device_id=(tgt,), device_id_type=pl.DeviceIdType.MESH,
                )
                rdma.start()
                rdma_list.append(rdma)
                sem_idx += 1


    fetches[2].wait()
    fetches[3].wait()
    wq = wq_vmem[...].astype(bf16)
    q = []
    for b in range(B):
        qb = lax.dot_general(
            x_vmem[b].astype(bf16), wq,
            (((1,), (0,)), ((), ())), preferred_element_type=jnp.float32,
        )
        q.append((qb * 0.125).astype(bf16))

    KW = SKV + 2 * HALO
    i2 = lax.broadcasted_iota(jnp.int32, (SQ, KW), 0)
    j2 = lax.broadcasted_iota(jnp.int32, (SQ, KW), 1)
    ki = pos * SKV - HALO + j2
    mask = (jnp.abs(i2 + HALO - j2) <= WIN) & (ki >= 0) & (ki < N_DEV * SKV)
    bias = jnp.where(mask, 0.0, -1e9).astype(jnp.float32)

    fetches[4].wait()
    wo = wo_vmem[...].astype(bf16)

    out_copies = []
    for b in range(B):
        for rdma in k_rdmas[2 * b:2 * b + 2]:
            rdma.wait_recv()
        ws = []
        for h in range(HQ):
            rows = slice(h * DH, (h + 1) * DH)
            khT = jnp.concatenate(
                [kleft[b][rows], kt_bf[2 * b][rows],
                 kt_bf[2 * b + 1][rows], kright[b][rows]], axis=1)
            s = lax.dot_general(
                q[b][:, rows], khT, (((1,), (0,)), ((), ())),
                preferred_element_type=jnp.float32,
            ) + bias
            e = jnp.exp(s)
            ws.append((e.astype(bf16),
                       1.0 / jnp.sum(e, axis=1, keepdims=True)))
        for rdma in v_rdmas[2 * b:2 * b + 2]:
            rdma.wait_recv()
        ctx_heads = []
        for h in range(HQ):
            rows = slice(h * DH, (h + 1) * DH)
            vhT = jnp.concatenate(
                [vleft[b][rows], vt_bf[2 * b][rows],
                 vt_bf[2 * b + 1][rows], vright[b][rows]], axis=1)
            e, recip = ws[h]
            ctx_heads.append(lax.dot_general(
                e, vhT, (((1,), (1,)), ((), ())),
                preferred_element_type=jnp.float32,
            ) * recip)
        ctx_b = jnp.concatenate(ctx_heads, axis=1).astype(bf16)
        out_vmem[b] = lax.dot_general(
            ctx_b, wo,
            (((1,), (0,)), ((), ())), preferred_element_type=jnp.float32,
        ).astype(bf16)
        oc = pltpu.make_async_copy(
            out_vmem.at[b], out_hbm.at[b], out_sems.at[b])
        oc.start()
        out_copies.append(oc)

    for rdma in k_rdmas + v_rdmas:
        rdma.wait_send()
    for oc in out_copies:
        oc.wait()


def kernel(x, Wq, K_ext, V_ext, Wo):
    kt = jnp.transpose(K_ext, (0, 2, 3, 1))
    vt = jnp.transpose(V_ext, (0, 2, 3, 1))
    hbm = pltpu.MemorySpace.HBM
    args = [pltpu.with_memory_space_constraint(a, hbm)
            for a in (x, Wq, kt, vt, Wo)]
    halo = pltpu.VMEM((B, HD, HALO), jnp.bfloat16)
    return pl.pallas_call(
        _body,
        out_shape=jax.ShapeDtypeStruct((B, SQ, D), jnp.bfloat16),
        in_specs=[pl.BlockSpec(memory_space=hbm)] * 5,
        out_specs=pl.BlockSpec(memory_space=hbm),
        scratch_shapes=[
            pltpu.VMEM((B, SQ, D), jnp.float32),
            pltpu.VMEM((D, HD), jnp.float32),
            pltpu.VMEM((B, HQ, DH, SKV), jnp.float32),
            pltpu.VMEM((B, HQ, DH, SKV), jnp.float32),
            pltpu.VMEM((HD, D), jnp.float32),
            pltpu.VMEM((B, SQ, D), jnp.bfloat16),
            pltpu.VMEM((B * 2, HD, HALO), jnp.bfloat16),
            pltpu.VMEM((B * 2, HD, HALO), jnp.bfloat16),
            halo, halo, halo, halo,
            pltpu.SemaphoreType.DMA((8,)),
            pltpu.SemaphoreType.DMA((8,)),
            pltpu.SemaphoreType.DMA((5,)),
            pltpu.SemaphoreType.DMA((B,)),
        ],
        compiler_params=pltpu.CompilerParams(collective_id=0),
    )(*args)
